# baseline (speedup 1.0000x reference)
import jax
import jax.numpy as jnp
from jax.experimental import pallas as pl

P = 16
E = 16
C = 96
FF = 96
EPS = 1e-6


def _moe_row(x_ref, g_ref, wr_ref, w1_ref, w2_ref, y_ref):
    g = g_ref[:]
    wr = wr_ref[:]
    wp = x_ref.shape[-1] // P

    for j in range(wp):
        xc = x_ref[0, :, :, j * P:(j + 1) * P].reshape(C, P * P)
        ms = jnp.mean(xc * xc, axis=0, keepdims=True)
        xn = xc * jax.lax.rsqrt(ms + EPS) * g
        pooled = jnp.mean(xn, axis=1, keepdims=True)
        logits = jax.lax.dot_general(
            pooled, wr, (((0,), (0,)), ((), ())),
            preferred_element_type=jnp.float32)
        lmax = jnp.max(logits)
        lane = jax.lax.broadcasted_iota(jnp.int32, (1, E), 1)
        idx = jnp.min(jnp.where(logits >= lmax, lane, E))
        w1 = w1_ref[idx]
        w2 = w2_ref[idx]
        h = jax.lax.dot_general(
            w1, xn, (((1,), (0,)), ((), ())),
            preferred_element_type=jnp.float32)
        h = h * jax.nn.sigmoid(h)
        o = jax.lax.dot_general(
            w2, h, (((1,), (0,)), ((), ())),
            preferred_element_type=jnp.float32)
        y_ref[0, :, :, j * P:(j + 1) * P] = (xc + o).reshape(C, P, P)


def kernel(x, g, Wr, W1, W2):
    B, Cc, H, W = x.shape
    Hp = H // P

    w1t = W1.transpose(0, 2, 1)
    w2t = W2.transpose(0, 2, 1)
    g2 = g.reshape(Cc, 1)

    y = pl.pallas_call(
        _moe_row,
        grid=(B, Hp),
        in_specs=[
            pl.BlockSpec((1, Cc, P, W), lambda b, i: (b, 0, i, 0)),
            pl.BlockSpec((Cc, 1), lambda b, i: (0, 0)),
            pl.BlockSpec((Cc, E), lambda b, i: (0, 0)),
            pl.BlockSpec((E, FF, Cc), lambda b, i: (0, 0, 0)),
            pl.BlockSpec((E, Cc, FF), lambda b, i: (0, 0, 0)),
        ],
        out_specs=pl.BlockSpec((1, Cc, P, W), lambda b, i: (b, 0, i, 0)),
        out_shape=jax.ShapeDtypeStruct((B, Cc, H, W), x.dtype),
    )(x, g2, Wr, w1t, w2t)

    return y


# speedup vs baseline: 2.2553x; 2.2553x over previous
import jax
import jax.numpy as jnp
from jax.experimental import pallas as pl
from jax.experimental.pallas import tpu as pltpu

P = 16
E = 16
C = 96
FF = 96
EPS = 1e-6


def _moe_row(x_ref, g_ref, wr_ref, w1_ref, w2_ref, y_ref, xt_ref, ot_ref):
    g = g_ref[:]
    wr = wr_ref[:]
    wp = x_ref.shape[-1] // P

    for r in range(P):
        xt_ref[r] = x_ref[0, :, r, :].T

    for j in range(wp):
        xp = xt_ref[:, j * P:(j + 1) * P, :]
        xp = xp.reshape(P * P, C)
        ms = jnp.mean(xp * xp, axis=1, keepdims=True)
        xn = xp * jax.lax.rsqrt(ms + EPS) * g
        pooled = jnp.mean(xn, axis=0, keepdims=True)
        logits = jax.lax.dot_general(
            pooled, wr, (((1,), (0,)), ((), ())),
            preferred_element_type=jnp.float32)
        lmax = jnp.max(logits)
        lane = jax.lax.broadcasted_iota(jnp.int32, (1, E), 1)
        idx = jnp.min(jnp.where(logits >= lmax, lane, E))
        w1 = w1_ref[idx]
        w2 = w2_ref[idx]
        h = jax.lax.dot_general(
            xn, w1, (((1,), (0,)), ((), ())),
            preferred_element_type=jnp.float32)
        h = h * jax.nn.sigmoid(h)
        o = jax.lax.dot_general(
            h, w2, (((1,), (0,)), ((), ())),
            preferred_element_type=jnp.float32)
        ot_ref[:, j * P:(j + 1) * P, :] = o.reshape(P, P, C)

    for r in range(P):
        y_ref[0, :, r, :] = x_ref[0, :, r, :] + ot_ref[r].T


def kernel(x, g, Wr, W1, W2):
    B, Cc, H, W = x.shape
    Hp = H // P

    y = pl.pallas_call(
        _moe_row,
        grid=(B, Hp),
        in_specs=[
            pl.BlockSpec((1, Cc, P, W), lambda b, i: (b, 0, i, 0)),
            pl.BlockSpec((1, Cc), lambda b, i: (0, 0)),
            pl.BlockSpec((Cc, E), lambda b, i: (0, 0)),
            pl.BlockSpec((E, Cc, FF), lambda b, i: (0, 0, 0)),
            pl.BlockSpec((E, FF, Cc), lambda b, i: (0, 0, 0)),
        ],
        out_specs=pl.BlockSpec((1, Cc, P, W), lambda b, i: (b, 0, i, 0)),
        out_shape=jax.ShapeDtypeStruct((B, Cc, H, W), x.dtype),
        scratch_shapes=[
            pltpu.VMEM((P, W, Cc), jnp.float32),
            pltpu.VMEM((P, W, Cc), jnp.float32),
        ],
    )(x, g.reshape(1, Cc), Wr, W1, W2)

    return y


# speedup vs baseline: 4.1285x; 1.8305x over previous
import jax
import jax.numpy as jnp
from jax.experimental import pallas as pl
from jax.experimental.pallas import tpu as pltpu

P = 16
E = 16
C = 96
FF = 96
EPS = 1e-6


def _moe_row(x_ref, g_ref, wr_ref, w1_ref, w2_ref, y_ref, xt_ref, ot_ref):
    g = g_ref[:]
    wr = wr_ref[:]
    w = x_ref.shape[-1]
    wp = w // P

    s = jnp.zeros((C, w), dtype=jnp.float32)
    for r in range(P):
        xr = x_ref[0, :, r, :]
        ms = jnp.mean(xr * xr, axis=0, keepdims=True)
        z = xr * jax.lax.rsqrt(ms + EPS) * g
        xt_ref[r] = z.T
        s = s + z

    wi = jax.lax.broadcasted_iota(jnp.int32, (w, wp), 0)
    ji = jax.lax.broadcasted_iota(jnp.int32, (w, wp), 1)
    seg = jnp.where(wi // P == ji, 1.0, 0.0)
    pooled = jax.lax.dot_general(
        s, seg, (((1,), (0,)), ((), ())),
        preferred_element_type=jnp.float32)
    logits = jax.lax.dot_general(
        pooled, wr, (((0,), (0,)), ((), ())),
        preferred_element_type=jnp.float32)
    lmax = jnp.max(logits, axis=1, keepdims=True)
    lane = jax.lax.broadcasted_iota(jnp.int32, (wp, E), 1)
    idx = jnp.min(jnp.where(logits >= lmax, lane, E), axis=1,
                  keepdims=True)

    for j in range(wp):
        e = idx[j, 0]
        xp = xt_ref[:, j * P:(j + 1) * P, :].reshape(P * P, C)
        w1 = w1_ref[e]
        w2 = w2_ref[e]
        h = jax.lax.dot_general(
            xp, w1, (((1,), (0,)), ((), ())),
            preferred_element_type=jnp.float32)
        h = h * jax.nn.sigmoid(h)
        o = jax.lax.dot_general(
            h, w2, (((1,), (0,)), ((), ())),
            preferred_element_type=jnp.float32)
        ot_ref[:, j * P:(j + 1) * P, :] = o.reshape(P, P, C)

    for r in range(P):
        y_ref[0, :, r, :] = x_ref[0, :, r, :] + ot_ref[r].T


def kernel(x, g, Wr, W1, W2):
    B, Cc, H, W = x.shape
    Hp = H // P

    y = pl.pallas_call(
        _moe_row,
        grid=(B, Hp),
        in_specs=[
            pl.BlockSpec((1, Cc, P, W), lambda b, i: (b, 0, i, 0)),
            pl.BlockSpec((Cc, 1), lambda b, i: (0, 0)),
            pl.BlockSpec((Cc, E), lambda b, i: (0, 0)),
            pl.BlockSpec((E, Cc, FF), lambda b, i: (0, 0, 0)),
            pl.BlockSpec((E, FF, Cc), lambda b, i: (0, 0, 0)),
        ],
        out_specs=pl.BlockSpec((1, Cc, P, W), lambda b, i: (b, 0, i, 0)),
        out_shape=jax.ShapeDtypeStruct((B, Cc, H, W), x.dtype),
        scratch_shapes=[
            pltpu.VMEM((P, W, Cc), jnp.float32),
            pltpu.VMEM((P, W, Cc), jnp.float32),
        ],
    )(x, g.reshape(Cc, 1), Wr, W1, W2)

    return y
